# baseline (speedup 1.0000x reference)
import jax
import jax.numpy as jnp
from jax import lax
from jax.experimental import pallas as pl
from jax.experimental.pallas import tpu as pltpu
from jax.experimental.pallas import tpu_sc as plsc

NUM_ACTIONS = 1000
PAD_ACTIONS = 1024
EMBED = 128
FEAT = 512
PACKED = FEAT // 2
GROUPS = FEAT // 32
BATCH = 16384

MLP_BM = 512

NC, NS = 2, 16
NW = NC * NS
BPW = BATCH // NW
CHUNK = 32
NCHUNK = BPW // CHUNK


def _mlp_body(t_ref, w1_ref, b1_ref, w2_ref, b2_ref, f_ref):
    e = t_ref[...].astype(jnp.bfloat16)
    h = jnp.dot(e, w1_ref[...].astype(jnp.bfloat16),
                preferred_element_type=jnp.float32)
    h = h + b1_ref[...]
    h = h * jax.nn.sigmoid(h)
    o = jnp.dot(h.astype(jnp.bfloat16), w2_ref[...].astype(jnp.bfloat16),
                preferred_element_type=jnp.float32)
    o = o + b2_ref[...]
    a = lax.bitcast_convert_type(
        o[:, :PACKED].astype(jnp.bfloat16), jnp.int16).astype(jnp.int32)
    b = lax.bitcast_convert_type(
        o[:, PACKED:].astype(jnp.bfloat16), jnp.int16).astype(jnp.int32)
    f_ref[...] = (a & jnp.int32(65535)) | (b << 16)


def _widen_chunk(src, dst):
    @plsc.parallel_loop(0, CHUNK, 1, unroll=4)
    def row_body(r):
        for g in range(PACKED // 16):
            v = src[r, pl.ds(g * 16, 16)]
            x = lax.bitcast_convert_type(v << 16, jnp.float32)
            y = lax.bitcast_convert_type(v & jnp.int32(-65536), jnp.float32)
            dst[r, pl.ds(g * 16, 16)] = x
            dst[r, pl.ds(PACKED + g * 16, 16)] = y


def _expand_body(feat_hbm, idx_hbm, out_hbm, idx_v, bg0, bg1, bf0, bf1,
                 gsem0, gsem1, ssem0, ssem1):
    wid = lax.axis_index("s") * NC + lax.axis_index("c")
    base = wid * BPW
    pltpu.sync_copy(idx_hbm.at[pl.ds(base, BPW)], idx_v)
    bgs = (bg0, bg1)
    bfs = (bf0, bf1)
    gsems = (gsem0, gsem1)
    ssems = (ssem0, ssem1)
    gathers = [None, None]
    scatters = [None, None]
    gathers[0] = pltpu.async_copy(
        feat_hbm.at[idx_v.at[pl.ds(0, CHUNK)]], bg0, gsem0)
    for j in range(NCHUNK):
        cur = j % 2
        nxt = (j + 1) % 2
        if j + 1 < NCHUNK:
            gathers[nxt] = pltpu.async_copy(
                feat_hbm.at[idx_v.at[pl.ds((j + 1) * CHUNK, CHUNK)]],
                bgs[nxt], gsems[nxt])
        gathers[cur].wait()
        if scatters[cur] is not None:
            scatters[cur].wait()
            scatters[cur] = None
        _widen_chunk(bgs[cur], bfs[cur])
        scatters[cur] = pltpu.async_copy(
            bfs[cur], out_hbm.at[pl.ds(base + j * CHUNK, CHUNK)],
            ssems[cur])
    for s in scatters:
        if s is not None:
            s.wait()


_expand = pl.kernel(
    _expand_body,
    out_type=jax.ShapeDtypeStruct((BATCH, FEAT), jnp.float32),
    mesh=plsc.VectorSubcoreMesh(core_axis_name="c", subcore_axis_name="s"),
    scratch_types=[
        pltpu.VMEM((BPW,), jnp.int32),
        pltpu.VMEM((CHUNK, PACKED), jnp.int32),
        pltpu.VMEM((CHUNK, PACKED), jnp.int32),
        pltpu.VMEM((CHUNK, FEAT), jnp.float32),
        pltpu.VMEM((CHUNK, FEAT), jnp.float32),
        pltpu.SemaphoreType.DMA,
        pltpu.SemaphoreType.DMA,
        pltpu.SemaphoreType.DMA,
        pltpu.SemaphoreType.DMA,
    ],
)


def kernel(action_indices, emb_table, W1, b1, W2, b2):
    feat = pl.pallas_call(
        _mlp_body,
        grid=(PAD_ACTIONS // MLP_BM,),
        in_specs=[
            pl.BlockSpec((MLP_BM, EMBED), lambda i: (i, 0)),
            pl.BlockSpec((EMBED, FEAT), lambda i: (0, 0)),
            pl.BlockSpec((1, FEAT), lambda i: (0, 0)),
            pl.BlockSpec((FEAT, FEAT), lambda i: (0, 0)),
            pl.BlockSpec((1, FEAT), lambda i: (0, 0)),
        ],
        out_specs=pl.BlockSpec((MLP_BM, PACKED), lambda i: (i, 0)),
        out_shape=jax.ShapeDtypeStruct((PAD_ACTIONS, PACKED), jnp.int32),
    )(emb_table, W1, b1.reshape(1, FEAT), W2, b2.reshape(1, FEAT))
    return _expand(feat, action_indices.astype(jnp.int32))


# speedup vs baseline: 1.1694x; 1.1694x over previous
import jax
import jax.numpy as jnp
from jax import lax
from jax.experimental import pallas as pl
from jax.experimental.pallas import tpu as pltpu
from jax.experimental.pallas import tpu_sc as plsc

NUM_ACTIONS = 1000
PAD_ACTIONS = 1024
EMBED = 128
FEAT = 512
PACKED = FEAT // 2
GROUPS = FEAT // 32
BATCH = 16384

MLP_BM = 512

NC, NS = 2, 16
NW = NC * NS
BPW = BATCH // NW
CHUNK = 64
NCHUNK = BPW // CHUNK


def _mlp_body(t_ref, w1_ref, b1_ref, w2_ref, b2_ref, f_ref):
    e = t_ref[...].astype(jnp.bfloat16)
    h = jnp.dot(e, w1_ref[...].astype(jnp.bfloat16),
                preferred_element_type=jnp.float32)
    h = h + b1_ref[...]
    h = h * jax.nn.sigmoid(h)
    o = jnp.dot(h.astype(jnp.bfloat16), w2_ref[...].astype(jnp.bfloat16),
                preferred_element_type=jnp.float32)
    o = o + b2_ref[...]
    a = lax.bitcast_convert_type(
        o[:, :PACKED].astype(jnp.bfloat16), jnp.int16).astype(jnp.int32)
    b = lax.bitcast_convert_type(
        o[:, PACKED:].astype(jnp.bfloat16), jnp.int16).astype(jnp.int32)
    f_ref[...] = (a & jnp.int32(65535)) | (b << 16)


def _widen_chunk(src, dst):
    @plsc.parallel_loop(0, CHUNK, 1, unroll=4)
    def row_body(r):
        for g in range(PACKED // 16):
            v = src[r, pl.ds(g * 16, 16)]
            x = lax.bitcast_convert_type(v << 16, jnp.float32)
            y = lax.bitcast_convert_type(v & jnp.int32(-65536), jnp.float32)
            dst[r, pl.ds(g * 16, 16)] = x
            dst[r, pl.ds(PACKED + g * 16, 16)] = y


def _expand_body(feat_hbm, idx_hbm, out_hbm, idx_v, bg0, bg1, bf0, bf1,
                 gsem0, gsem1, ssem0, ssem1):
    wid = lax.axis_index("s") * NC + lax.axis_index("c")
    base = wid * BPW
    pltpu.sync_copy(idx_hbm.at[pl.ds(base, BPW)], idx_v)
    bgs = (bg0, bg1)
    bfs = (bf0, bf1)
    gsems = (gsem0, gsem1)
    ssems = (ssem0, ssem1)

    def gather(c, b):
        pltpu.async_copy(
            feat_hbm.at[idx_v.at[pl.ds(c * CHUNK, CHUNK)]], bgs[b], gsems[b])

    def gather_wait(c, b):
        pltpu.make_async_copy(
            feat_hbm.at[idx_v.at[pl.ds(c * CHUNK, CHUNK)]], bgs[b],
            gsems[b]).wait()

    def scatter(c, b):
        pltpu.async_copy(
            bfs[b], out_hbm.at[pl.ds(base + c * CHUNK, CHUNK)], ssems[b])

    def scatter_wait(c, b):
        pltpu.make_async_copy(
            bfs[b], out_hbm.at[pl.ds(base + c * CHUNK, CHUNK)],
            ssems[b]).wait()

    gather(0, 0)
    gather(1, 1)

    def pair_body(k, carry):
        for b in range(2):
            c = 2 * k + b
            gather_wait(c, b)

            @pl.when(k > 0)
            def _():
                scatter_wait(c - 2, b)
            _widen_chunk(bgs[b], bfs[b])
            scatter(c, b)

            @pl.when(k < NCHUNK // 2 - 1)
            def _():
                gather(c + 2, b)
        return carry

    lax.fori_loop(0, NCHUNK // 2, pair_body, 0)
    for b in range(2):
        scatter_wait(NCHUNK - 2 + b, b)


_expand = pl.kernel(
    _expand_body,
    out_type=jax.ShapeDtypeStruct((BATCH, FEAT), jnp.float32),
    mesh=plsc.VectorSubcoreMesh(core_axis_name="c", subcore_axis_name="s"),
    scratch_types=[
        pltpu.VMEM((BPW,), jnp.int32),
        pltpu.VMEM((CHUNK, PACKED), jnp.int32),
        pltpu.VMEM((CHUNK, PACKED), jnp.int32),
        pltpu.VMEM((CHUNK, FEAT), jnp.float32),
        pltpu.VMEM((CHUNK, FEAT), jnp.float32),
        pltpu.SemaphoreType.DMA,
        pltpu.SemaphoreType.DMA,
        pltpu.SemaphoreType.DMA,
        pltpu.SemaphoreType.DMA,
    ],
)


def kernel(action_indices, emb_table, W1, b1, W2, b2):
    feat = pl.pallas_call(
        _mlp_body,
        grid=(PAD_ACTIONS // MLP_BM,),
        in_specs=[
            pl.BlockSpec((MLP_BM, EMBED), lambda i: (i, 0)),
            pl.BlockSpec((EMBED, FEAT), lambda i: (0, 0)),
            pl.BlockSpec((1, FEAT), lambda i: (0, 0)),
            pl.BlockSpec((FEAT, FEAT), lambda i: (0, 0)),
            pl.BlockSpec((1, FEAT), lambda i: (0, 0)),
        ],
        out_specs=pl.BlockSpec((MLP_BM, PACKED), lambda i: (i, 0)),
        out_shape=jax.ShapeDtypeStruct((PAD_ACTIONS, PACKED), jnp.int32),
    )(emb_table, W1, b1.reshape(1, FEAT), W2, b2.reshape(1, FEAT))
    return _expand(feat, action_indices.astype(jnp.int32))
